# baseline (speedup 1.0000x reference)
import functools

import jax
import jax.numpy as jnp
from jax import lax
from jax.experimental import pallas as pl
from jax.experimental.pallas import tpu as pltpu
from jax.experimental.pallas import tpu_sc as plsc

_NUM_CORES = 2
_NUM_SUBCORES = 16
_NUM_WORKERS = _NUM_CORES * _NUM_SUBCORES

_CHUNK = 640


@functools.cache
def _build_rowmajor(d: int, vocab: int):
    assert d == 64
    n_full = vocab // 128
    tail = vocab - n_full * 128
    assert tail == 64
    n_iters = (n_full + _NUM_WORKERS - 1) // _NUM_WORKERS
    n_extra = n_full % _NUM_WORKERS
    n_groups = (n_iters + 1) // 2

    mesh = plsc.VectorSubcoreMesh(core_axis_name="c", subcore_axis_name="s")

    @functools.partial(
        pl.kernel,
        out_type=jax.ShapeDtypeStruct((vocab // 2, 2 * d), jnp.float32),
        mesh=mesh,
        scratch_types=[
            pltpu.VMEM((d, 128), jnp.float32),
            pltpu.VMEM((d, 128), jnp.float32),
            pltpu.VMEM((d, 128), jnp.float32),
            pltpu.VMEM((d, 128), jnp.float32),
            pltpu.SemaphoreType.DMA,
            pltpu.SemaphoreType.DMA,
            pltpu.SemaphoreType.DMA,
            pltpu.SemaphoreType.DMA,
        ],
        compiler_params=pltpu.CompilerParams(needs_layout_passes=False),
    )
    def transpose_kernel(tab_t, tail_rows, out_hbm, slab0, slab1,
                         rows0, rows1, gi0, gi1, wo0, wo1):
        wid = lax.axis_index("s") * _NUM_CORES + lax.axis_index("c")
        slabs = (slab0, slab1)
        rows = (rows0, rows1)
        gsem = (gi0, gi1)
        wsem = (wo0, wo1)
        lane = lax.broadcasted_iota(jnp.int32, (16,), 0)
        vj = [16 * j + lane for j in range(d // 16)]
        n_w = jnp.where(wid < n_extra, n_iters, n_iters - 1)

        @pl.when(wid == 0)
        def _():
            pltpu.sync_copy(tail_rows, rows0.at[pl.ds(0, tail // 2)])
            pltpu.sync_copy(rows0.at[pl.ds(0, tail // 2)],
                            out_hbm.at[pl.ds(n_full * 64, tail // 2)])

        def start_in(s, b):
            @pl.when(s < n_full)
            def _():
                pltpu.async_copy(
                    tab_t.at[:, pl.ds(s * 128, 128)], slabs[b], gsem[b])

        def wait_in(b):
            pltpu.make_async_copy(
                tab_t.at[:, pl.ds(0, 128)], slabs[b], gsem[b]).wait()

        def transpose(b):
            def body_p(p, carry):
                base = 2 * p
                for q in range(2):
                    col = jnp.broadcast_to(base + q, (16,))
                    for j in range(d // 16):
                        vals = plsc.load_gather(slabs[b], [vj[j], col])
                        rows[b][p, pl.ds(q * d + 16 * j, 16)] = vals
                return carry

            lax.fori_loop(0, d, body_p, 0, unroll=2)

        start_in(wid, 0)
        start_in(wid + _NUM_WORKERS, 1)

        def group(g, carry):
            for b in range(2):
                t = 2 * g + b
                s = wid + t * _NUM_WORKERS

                @pl.when(s < n_full)
                def _():
                    wait_in(b)

                    @pl.when(t >= 2)
                    def _():
                        pltpu.make_async_copy(
                            rows[b], out_hbm.at[pl.ds(0, 64)],
                            wsem[b]).wait()

                    transpose(b)
                    start_in(s + 2 * _NUM_WORKERS, b)
                    is_last2 = t >= n_w - 2

                    @pl.when(is_last2)
                    def _():
                        pltpu.sync_copy(rows[b],
                                        out_hbm.at[pl.ds(s * 64, 64)])

                    @pl.when(jnp.logical_not(is_last2))
                    def _():
                        pltpu.async_copy(
                            rows[b], out_hbm.at[pl.ds(s * 64, 64)],
                            wsem[b])

            return carry

        lax.fori_loop(0, n_groups, group, 0, unroll=False)

    return transpose_kernel


@functools.cache
def _build_gather(n_total: int, vocab: int, d: int, dp: int):
    assert n_total % _NUM_WORKERS == 0
    n_per_w = n_total // _NUM_WORKERS
    chunk = min(_CHUNK, n_per_w)
    assert n_per_w % chunk == 0
    n_chunks = n_per_w // chunk
    assert n_chunks % 2 == 0 and n_chunks >= 2

    mesh = plsc.VectorSubcoreMesh(core_axis_name="c", subcore_axis_name="s")

    @functools.partial(
        pl.kernel,
        out_type=jax.ShapeDtypeStruct((n_total, dp), jnp.float32),
        mesh=mesh,
        scratch_types=[
            pltpu.VMEM((n_per_w,), jnp.int32),
            pltpu.VMEM((chunk, d), jnp.float32),
            pltpu.VMEM((chunk, d), jnp.float32),
            pltpu.SemaphoreType.DMA,
            pltpu.SemaphoreType.DMA,
            pltpu.SemaphoreType.DMA,
            pltpu.SemaphoreType.DMA,
        ],
        compiler_params=pltpu.CompilerParams(use_tc_tiling_on_sc=False),
    )
    def gather_kernel(idx_hbm, table_hbm, out_hbm, idx_v, rows0, rows1,
                      g0, g1, w0, w1):
        wid = lax.axis_index("s") * _NUM_CORES + lax.axis_index("c")
        base_w = wid * n_per_w
        rows = (rows0, rows1)
        gsem = (g0, g1)
        wsem = (w0, w1)

        pltpu.sync_copy(idx_hbm.at[pl.ds(base_w, n_per_w)], idx_v)

        def start_gather(j, b):
            pltpu.async_copy(
                table_hbm.at[idx_v.at[pl.ds(j * chunk, chunk)]],
                rows[b], gsem[b])

        def wait_gather(b):
            pltpu.make_async_copy(
                table_hbm.at[pl.ds(0, chunk)], rows[b], gsem[b]).wait()

        def out_window(i):
            return out_hbm.at[pl.ds(base_w + i * chunk, chunk), pl.ds(0, d)]

        start_gather(0, 0)
        start_gather(1, 1)

        def group(g, carry):
            for b in range(2):
                i = g * 2 + b
                wait_gather(b)
                wb = pltpu.make_async_copy(rows[b], out_window(i), wsem[b])
                wb.start()
                j = i + 2

                @pl.when(j < n_chunks)
                def _():
                    wb.wait()
                    start_gather(j, b)

            return carry

        lax.fori_loop(0, n_chunks // 2, group, 0, unroll=False)

        for b in range(2):
            i = n_chunks - 2 + b
            pltpu.make_async_copy(rows[b], out_window(i), wsem[b]).wait()

    return gather_kernel


def kernel(x, table):
    b, h = x.shape
    vocab, d = table.shape
    dp = 128
    n_full = vocab // 128
    tail_rows = table[n_full * 128:, :].reshape((vocab - n_full * 128) // 2,
                                                2 * d)
    table_rows = _build_rowmajor(d, vocab)(table.T, tail_rows)
    table_lin = table_rows.reshape(vocab, d)
    idx = x.reshape(b * h).astype(jnp.int32)
    out_p = _build_gather(b * h, vocab, d, dp)(idx, table_lin)
    return out_p[:, :d].reshape(b, h, d)


# speedup vs baseline: 1.9670x; 1.9670x over previous
import functools

import jax
import jax.numpy as jnp
from jax import lax
from jax.experimental import pallas as pl
from jax.experimental.pallas import tpu as pltpu
from jax.experimental.pallas import tpu_sc as plsc

_NUM_CORES = 2
_NUM_SUBCORES = 16
_NUM_WORKERS = _NUM_CORES * _NUM_SUBCORES

_CHUNK = 640


@functools.cache
def _build_gather(n_total: int, vocab: int, d: int, dp: int):
    assert n_total % _NUM_WORKERS == 0
    n_per_w = n_total // _NUM_WORKERS
    chunk = min(_CHUNK, n_per_w)
    assert n_per_w % chunk == 0
    n_chunks = n_per_w // chunk
    assert n_chunks % 2 == 0 and n_chunks >= 2

    mesh = plsc.VectorSubcoreMesh(core_axis_name="c", subcore_axis_name="s")

    @functools.partial(
        pl.kernel,
        out_type=jax.ShapeDtypeStruct((n_total, dp), jnp.float32),
        mesh=mesh,
        scratch_types=[
            pltpu.VMEM((n_per_w,), jnp.int32),
            pltpu.VMEM((chunk, d), jnp.float32),
            pltpu.VMEM((chunk, d), jnp.float32),
            pltpu.SemaphoreType.DMA,
            pltpu.SemaphoreType.DMA,
            pltpu.SemaphoreType.DMA,
            pltpu.SemaphoreType.DMA,
        ],
        compiler_params=pltpu.CompilerParams(use_tc_tiling_on_sc=False),
    )
    def gather_kernel(idx_hbm, table_hbm, out_hbm, idx_v, rows0, rows1,
                      g0, g1, w0, w1):
        wid = lax.axis_index("s") * _NUM_CORES + lax.axis_index("c")
        base_w = wid * n_per_w
        rows = (rows0, rows1)
        gsem = (g0, g1)
        wsem = (w0, w1)

        pltpu.sync_copy(idx_hbm.at[pl.ds(base_w, n_per_w)], idx_v)

        def start_gather(j, b):
            pltpu.async_copy(
                table_hbm.at[idx_v.at[pl.ds(j * chunk, chunk)]],
                rows[b], gsem[b])

        def wait_gather(b):
            pltpu.make_async_copy(
                table_hbm.at[pl.ds(0, chunk)], rows[b], gsem[b]).wait()

        def out_window(i):
            return out_hbm.at[pl.ds(base_w + i * chunk, chunk), pl.ds(0, d)]

        start_gather(0, 0)
        start_gather(1, 1)

        def group(g, carry):
            for b in range(2):
                i = g * 2 + b
                wait_gather(b)
                wb = pltpu.make_async_copy(rows[b], out_window(i), wsem[b])
                wb.start()
                j = i + 2

                @pl.when(j < n_chunks)
                def _():
                    wb.wait()
                    start_gather(j, b)

            return carry

        lax.fori_loop(0, n_chunks // 2, group, 0, unroll=False)

        for b in range(2):
            i = n_chunks - 2 + b
            pltpu.make_async_copy(rows[b], out_window(i), wsem[b]).wait()

    return gather_kernel


def kernel(x, table):
    b, h = x.shape
    vocab, d = table.shape
    dp = 128
    idx = x.reshape(b * h).astype(jnp.int32)
    out_p = _build_gather(b * h, vocab, d, dp)(idx, table)
    return out_p[:, :d].reshape(b, h, d)
